# baseline (speedup 1.0000x reference)
import functools

import jax
import jax.numpy as jnp
from jax.experimental import pallas as pl
from jax.experimental.pallas import tpu as pltpu

EPS = 1e-5
K = 20
NUM_NODE = 64
B = 16
N = 1024


def _lrelu(x):
    return jax.nn.leaky_relu(x, 0.2)


def _bn_host(x, axes):
    m = jnp.mean(x, axes, keepdims=True)
    v = jnp.var(x, axes, keepdims=True)
    return (x - m) * jax.lax.rsqrt(v + EPS)


def _dot(a, b):
    return jax.lax.dot_general(a, b, (((1,), (0,)), ((), ())),
                               preferred_element_type=jnp.float32,
                               precision=jax.lax.Precision.HIGHEST)


def _neg_dist(x):
    g = jax.lax.dot_general(x, x, (((0,), (0,)), ((), ())),
                            preferred_element_type=jnp.float32,
                            precision=jax.lax.Precision.HIGHEST)
    sq = jnp.sum(x * x, axis=0)
    return (2.0 * g - sq[None, :]) - sq[:, None]


def _topk_idx(cur):
    sub = jax.lax.broadcasted_iota(jnp.int32, (N, N), 0)
    rows = jax.lax.broadcasted_iota(jnp.int32, (K, N), 0)

    def body(t, carry):
        cur, idx_all = carry
        m = jnp.max(cur, axis=0)
        hit = cur == m[None, :]
        idx = jnp.min(jnp.where(hit, sub, N), axis=0)
        h = sub == idx[None, :]
        return (jnp.where(h, -jnp.inf, cur),
                jnp.where(rows == t, idx[None, :], idx_all))

    _, idx_all = jax.lax.fori_loop(
        0, K, body, (cur, jnp.zeros((K, N), jnp.int32)))
    return idx_all


def _edge_kernel(x_ref, wn_ref, wd_ref, m_ref, s1_ref, s2_ref, bv_ref):
    x = x_ref[0]
    idx_all = _topk_idx(_neg_dist(x))
    a = _dot(wn_ref[...], x)
    bv = _dot(wd_ref[...], x)
    sub = jax.lax.broadcasted_iota(jnp.int32, (N, N), 0)
    ssum = jnp.zeros((N, N), jnp.float32)
    accmax = jnp.full(a.shape, -jnp.inf, jnp.float32)
    for t in range(K):
        hf = (sub == idx_all[t][None, :]).astype(jnp.float32)
        ssum = ssum + hf
        accmax = jnp.maximum(accmax, _dot(a, hf))
    m_ref[0] = accmax
    s1_ref[0] = _dot(a, ssum)
    s2_ref[0] = _dot(a * a, ssum)
    bv_ref[0] = bv


def _edge_call(x, wn, wd):
    b, c, n = x.shape
    o = wn.shape[0]
    f32 = jnp.float32
    return pl.pallas_call(
        _edge_kernel,
        grid=(b,),
        in_specs=[
            pl.BlockSpec((1, c, n), lambda i: (i, 0, 0)),
            pl.BlockSpec((o, c), lambda i: (0, 0)),
            pl.BlockSpec((o, c), lambda i: (0, 0)),
        ],
        out_specs=[pl.BlockSpec((1, o, n), lambda i: (i, 0, 0))] * 4,
        out_shape=[jax.ShapeDtypeStruct((b, o, n), f32)] * 4,
        compiler_params=pltpu.CompilerParams(
            dimension_semantics=("parallel",),
            vmem_limit_bytes=100 * 1024 * 1024,
        ),
    )(x, wn, wd)


def _pad_c(a, c):
    return jnp.pad(a, [(0, 0), (0, c - a.shape[1])] + [(0, 0)] * (a.ndim - 2))


def _edge_stage(x, w):
    c = x.shape[1]
    wn = w[:, :c]
    wd = w[:, c:] - w[:, :c]
    if c < 8:
        x, wn, wd = _pad_c(x, 8), _pad_c(wn, 8), _pad_c(wd, 8)
    m, s1, s2, bv = _edge_call(x, wn, wd)
    cnt = x.shape[0] * x.shape[2] * K
    tot = jnp.sum(s1 + K * bv, axis=(0, 2))
    totq = jnp.sum(s2 + 2.0 * bv * s1 + K * bv * bv, axis=(0, 2))
    mean = tot / cnt
    var = totq / cnt - mean * mean
    inv = jax.lax.rsqrt(var + EPS)
    return _lrelu((m + bv - mean[None, :, None]) * inv[None, :, None])


def _tnet1_kernel(x_ref, wn_ref, wd_ref, z_ref, s_ref):
    x = x_ref[0]
    idx_all = _topk_idx(_neg_dist(x))
    bx = _dot(wd_ref[...], x)
    sub = jax.lax.broadcasted_iota(jnp.int32, (N, N), 0)
    sv = jnp.zeros((64,), jnp.float32)
    qv = jnp.zeros((64,), jnp.float32)
    for t in range(K):
        hf = (sub == idx_all[t][None, :]).astype(jnp.float32)
        xt = _dot(x, hf)
        z = _dot(wn_ref[...], xt) + bx
        z_ref[0, t] = z
        sv = sv + jnp.sum(z, axis=1)
        qv = qv + jnp.sum(z * z, axis=1)
    s_ref[0, 0] = sv
    s_ref[0, 1] = qv


def _tnet2_kernel(z_ref, sc_ref, sh_ref, w2_ref, m_ref, s_ref):
    sc = sc_ref[...]
    sh = sh_ref[...]
    accmax = jnp.full((128, N), -jnp.inf, jnp.float32)
    sv = jnp.zeros((128,), jnp.float32)
    qv = jnp.zeros((128,), jnp.float32)
    for t in range(K):
        a = z_ref[0, t] * sc + sh
        a = jnp.where(a > 0, a, 0.2 * a)
        z2 = _dot(w2_ref[...], a)
        accmax = jnp.maximum(accmax, z2)
        sv = sv + jnp.sum(z2, axis=1)
        qv = qv + jnp.sum(z2 * z2, axis=1)
    m_ref[0] = accmax
    s_ref[0, 0] = sv
    s_ref[0, 1] = qv


def _convmax_kernel(x_ref, w_ref, m_ref, s_ref):
    z = _dot(w_ref[...], x_ref[0])
    m_ref[0, 0] = jnp.max(z, axis=1)
    s_ref[0, 0] = jnp.sum(z, axis=1)
    s_ref[0, 1] = jnp.sum(z * z, axis=1)


def _convmax(x, w):
    b, c, n = x.shape
    o = w.shape[0]
    f32 = jnp.float32
    m, s = pl.pallas_call(
        _convmax_kernel,
        grid=(b,),
        in_specs=[
            pl.BlockSpec((1, c, n), lambda i: (i, 0, 0)),
            pl.BlockSpec((o, c), lambda i: (0, 0)),
        ],
        out_specs=[
            pl.BlockSpec((1, 1, o), lambda i: (i, 0, 0)),
            pl.BlockSpec((1, 2, o), lambda i: (i, 0, 0)),
        ],
        out_shape=[
            jax.ShapeDtypeStruct((b, 1, o), f32),
            jax.ShapeDtypeStruct((b, 2, o), f32),
        ],
        compiler_params=pltpu.CompilerParams(
            dimension_semantics=("parallel",),
            vmem_limit_bytes=100 * 1024 * 1024,
        ),
    )(x, w)
    cnt = b * n
    mean = jnp.sum(s[:, 0], axis=0) / cnt
    var = jnp.sum(s[:, 1], axis=0) / cnt - mean * mean
    inv = jax.lax.rsqrt(var + EPS)
    return _lrelu((m[:, 0] - mean[None, :]) * inv[None, :])


def _transform_net(x, p):
    f32 = jnp.float32
    w1 = p['t_conv1_w']
    wn1 = _pad_c(w1[:, :3], 8)
    wd1 = _pad_c(w1[:, 3:] - w1[:, :3], 8)
    xp = _pad_c(x, 8)
    z1, s1 = pl.pallas_call(
        _tnet1_kernel,
        grid=(B,),
        in_specs=[
            pl.BlockSpec((1, 8, N), lambda i: (i, 0, 0)),
            pl.BlockSpec((64, 8), lambda i: (0, 0)),
            pl.BlockSpec((64, 8), lambda i: (0, 0)),
        ],
        out_specs=[
            pl.BlockSpec((1, K, 64, N), lambda i: (i, 0, 0, 0)),
            pl.BlockSpec((1, 2, 64), lambda i: (i, 0, 0)),
        ],
        out_shape=[
            jax.ShapeDtypeStruct((B, K, 64, N), f32),
            jax.ShapeDtypeStruct((B, 2, 64), f32),
        ],
        compiler_params=pltpu.CompilerParams(
            dimension_semantics=("parallel",),
            vmem_limit_bytes=100 * 1024 * 1024,
        ),
    )(xp, wn1, wd1)
    cnt = B * N * K
    mean = jnp.sum(s1[:, 0], axis=0) / cnt
    var = jnp.sum(s1[:, 1], axis=0) / cnt - mean * mean
    inv = jax.lax.rsqrt(var + EPS)
    sc = inv.reshape(64, 1)
    sh = (-mean * inv).reshape(64, 1)
    m2, s2 = pl.pallas_call(
        _tnet2_kernel,
        grid=(B,),
        in_specs=[
            pl.BlockSpec((1, K, 64, N), lambda i: (i, 0, 0, 0)),
            pl.BlockSpec((64, 1), lambda i: (0, 0)),
            pl.BlockSpec((64, 1), lambda i: (0, 0)),
            pl.BlockSpec((128, 64), lambda i: (0, 0)),
        ],
        out_specs=[
            pl.BlockSpec((1, 128, N), lambda i: (i, 0, 0)),
            pl.BlockSpec((1, 2, 128), lambda i: (i, 0, 0)),
        ],
        out_shape=[
            jax.ShapeDtypeStruct((B, 128, N), f32),
            jax.ShapeDtypeStruct((B, 2, 128), f32),
        ],
        compiler_params=pltpu.CompilerParams(
            dimension_semantics=("parallel",),
            vmem_limit_bytes=100 * 1024 * 1024,
        ),
    )(z1, sc, sh, p['t_conv2_w'])
    mean2 = jnp.sum(s2[:, 0], axis=0) / cnt
    var2 = jnp.sum(s2[:, 1], axis=0) / cnt - mean2 * mean2
    inv2 = jax.lax.rsqrt(var2 + EPS)
    a2 = _lrelu((m2 - mean2[None, :, None]) * inv2[None, :, None])
    h = _convmax(a2, p['t_conv3_w'])
    h = _lrelu(_bn_host(h @ p['t_fc1_w'].T, (0,)))
    h = _lrelu(_bn_host(h @ p['t_fc2_w'].T + p['t_fc2_b'], (0,)))
    h = h @ p['t_fc3_w'].T + p['t_fc3_b']
    return h.reshape(-1, 3, 3) + jnp.eye(3, dtype=h.dtype)


def _gather_pts(x, idx):
    return jnp.take_along_axis(x, idx[:, None, :], axis=2)


def _group_gather(x, idx):
    return jax.vmap(lambda xb, ib: xb[:, ib])(x, idx)


def _knn_query(pts, q, k):
    sp = jnp.sum(pts * pts, axis=1)
    sq = jnp.sum(q * q, axis=1)
    neg = 2.0 * jnp.einsum('bcm,bcn->bmn', q, pts) - sq[:, :, None] - sp[:, None, :]
    return jax.lax.top_k(neg, k)[1]


def _fps(loc, m):
    pts = loc.transpose(0, 2, 1)

    def one(p):
        n = p.shape[0]

        def step(carry, _):
            d, last = carry
            nd = jnp.sum((p - p[last]) ** 2, axis=-1)
            d = jnp.minimum(d, nd)
            nxt = jnp.argmax(d).astype(jnp.int32)
            return (d, nxt), nxt

        init = (jnp.full((n,), 1e10, p.dtype), jnp.int32(0))
        _, rest = jax.lax.scan(step, init, None, length=m - 1)
        return jnp.concatenate([jnp.zeros((1,), jnp.int32), rest])

    return jax.vmap(one)(pts)


def _conv_block_host(x, w, bias=None, act='leakyrelu'):
    y = jnp.einsum('oc,bc...->bo...', w, x)
    if bias is not None:
        y = y + bias.reshape((1, -1) + (1,) * (y.ndim - 2))
    y = _bn_host(y, (0,) + tuple(range(2, y.ndim)))
    return _lrelu(y) if act == 'leakyrelu' else jax.nn.relu(y)


def _adapt_layer(fea, loc, p):
    fidx = _fps(loc, NUM_NODE)
    floc = _gather_pts(loc, fidx)
    ffea = _gather_pts(fea, fidx)
    gidx = _knn_query(loc, floc, NUM_NODE)
    gfea = _group_gather(fea, gidx) - ffea[:, :, :, None]
    seman = jnp.tanh(jnp.einsum('oc,bcmk->bomk', p['off_w'], gfea))
    gloc = _group_gather(loc, gidx) - floc[:, :, :, None]
    node_off = jnp.mean(seman * gloc, axis=-1)
    node_loc = floc + node_off
    gidx2 = _knn_query(loc, node_loc, NUM_NODE)
    node_pool = jnp.max(_group_gather(fea, gidx2), axis=-1)
    h1 = _conv_block_host(node_pool, p['trans_w'], p['trans_b'], act='relu')
    h2 = _conv_block_host(ffea, p['res_w'], p['res_b'], act='relu')
    node_fea = h1 + h2
    sp = jnp.sum(loc * loc, axis=1)
    sn = jnp.sum(node_loc * node_loc, axis=1)
    d = sp[:, :, None] - 2.0 * jnp.einsum('bcn,bcm->bnm', loc, node_loc) + sn[:, None, :]
    negd, ni = jax.lax.top_k(-d, 3)
    w = 1.0 / (jnp.maximum(-negd, 0.0) + 1e-8)
    w = w / jnp.sum(w, axis=-1, keepdims=True)
    nbrf = _group_gather(node_pool, ni)
    interp = jnp.sum(nbrf * w[:, None, :, :], axis=-1)
    return fea + interp, node_fea, node_off


def _head(x, p, tag):
    h = _lrelu(_bn_host(x @ p[tag + '_w1'].T + p[tag + '_b1'], (0,)))
    h = _lrelu(_bn_host(h @ p[tag + '_w2'].T + p[tag + '_b2'], (0,)))
    return h @ p[tag + '_w3'].T + p[tag + '_b3']


@jax.jit
def _forward(x, p):
    x = x[..., 0]
    xyz = x
    t = _transform_net(x, p)
    x = jnp.einsum('bij,bjn->bin', t, x)
    x1 = _edge_stage(x, p['conv1_w'])
    x2 = _edge_stage(x1, p['conv2_w'])
    x3 = _edge_stage(x2, p['conv3_w'])
    x4 = _edge_stage(x3, p['conv4_w'])
    xcat = jnp.concatenate([x1, x2, x3, x4], axis=1)
    x_fea, _, _ = _adapt_layer(xcat, xyz, p)
    x6 = _convmax(x_fea, p['conv6_w'])
    return _head(x6, p, 'c1'), _head(x6, p, 'c2')


def kernel(x, params):
    return _forward(x, params)
